# baseline (speedup 1.0000x reference)
import functools

import jax
import jax.numpy as jnp
from jax import lax
from jax.experimental import pallas as pl
from jax.experimental.pallas import tpu as pltpu
from jax.experimental.pallas import tpu_sc as plsc

B = 16384
N_SPARSE = 26
DENSE_DIMS = 13
VOCAB = 100000
NC = 2
NSUB = 16
NW = NC * NSUB
RPW = B // NW
LANES = 16
CHUNKS = RPW // LANES

def _fieldwise_sc_body(raw_hbm, table_hbm, w_hbm, out_hbm,
                       slab, idxb, vals, outv, wv, sem):
    wid = lax.axis_index("s") * NC + lax.axis_index("c")
    base = wid * RPW
    ncol = N_SPARSE + DENSE_DIMS
    pltpu.sync_copy(raw_hbm.at[pl.ds(base * ncol, RPW * ncol)], slab)
    pltpu.sync_copy(w_hbm, wv)
    lane = lax.iota(jnp.int32, LANES)
    wvecs = [wv[pl.ds(d * LANES, LANES)] for d in range(DENSE_DIMS)]

    def build(r, carry):
        flat = (r * LANES + lane) * (N_SPARSE + DENSE_DIMS)
        for f in range(N_SPARSE):
            col = plsc.load_gather(slab, [flat + f])
            gidx = col.astype(jnp.int32) + f * VOCAB
            idxb[pl.ds(f * RPW + r * LANES, LANES)] = gidx
        return carry

    lax.fori_loop(0, CHUNKS, build, 0)

    pltpu.async_copy(table_hbm.at[idxb], vals, sem).wait()

    def accum(r, carry):
        flat = (r * LANES + lane) * (N_SPARSE + DENSE_DIMS)
        acc = vals[pl.ds(r * LANES, LANES)]
        for f in range(1, N_SPARSE):
            acc = acc + vals[pl.ds(f * RPW + r * LANES, LANES)]
        for d in range(DENSE_DIMS):
            col = plsc.load_gather(slab, [flat + (N_SPARSE + d)])
            acc = acc + col * wvecs[d]
        outv[pl.ds(r * LANES, LANES)] = acc
        return carry

    lax.fori_loop(0, CHUNKS, accum, 0)
    pltpu.sync_copy(outv, out_hbm.at[pl.ds(base, RPW)])


@functools.cache
def _build_sc_kernel():
    mesh = plsc.VectorSubcoreMesh(
        core_axis_name="c", subcore_axis_name="s",
        num_cores=NC, num_subcores=NSUB)
    return pl.kernel(
        _fieldwise_sc_body,
        out_type=jax.ShapeDtypeStruct((B,), jnp.float32),
        mesh=mesh,
        compiler_params=pltpu.CompilerParams(needs_layout_passes=False),
        scratch_types=[
            pltpu.VMEM((RPW * (N_SPARSE + DENSE_DIMS),), jnp.float32),
            pltpu.VMEM((N_SPARSE * RPW,), jnp.int32),
            pltpu.VMEM((N_SPARSE * RPW,), jnp.float32),
            pltpu.VMEM((RPW,), jnp.float32),
            pltpu.VMEM((DENSE_DIMS * LANES,), jnp.float32),
            pltpu.SemaphoreType.DMA,
        ],
    )


def kernel(raw_feat, sparse_tables, W_dense):
    table_flat = sparse_tables.reshape(N_SPARSE * VOCAB)
    raw_flat = raw_feat.reshape(B * (N_SPARSE + DENSE_DIMS))
    w_rep = jnp.repeat(W_dense[0].astype(jnp.float32), LANES)
    return _build_sc_kernel()(raw_flat, table_flat, w_rep)


# speedup vs baseline: 2.5082x; 2.5082x over previous
import functools

import jax
import jax.numpy as jnp
from jax import lax
from jax.experimental import pallas as pl
from jax.experimental.pallas import tpu as pltpu
from jax.experimental.pallas import tpu_sc as plsc

B = 16384
N_SPARSE = 26
DENSE_DIMS = 13
NCOL = N_SPARSE + DENSE_DIMS
VOCAB = 100000
NC = 2
NSUB = 16
NW = NC * NSUB
RPW = B // NW
LANES = 16
CHUNKS = RPW // LANES
SP_CHUNKS = N_SPARSE * CHUNKS


def _fieldwise_sc_body(raw_hbm, table_hbm, w_hbm, out_hbm,
                       slab, idxb, vals, outv, wv, sem):
    wid = lax.axis_index("s") * NC + lax.axis_index("c")
    base = wid * RPW
    descs = [
        pltpu.async_copy(raw_hbm.at[pl.ds(f * B + base, RPW)],
                         slab.at[pl.ds(f * RPW, RPW)], sem)
        for f in range(NCOL)
    ]
    pltpu.sync_copy(w_hbm, wv)
    for d in descs:
        d.wait()

    def build(c, carry):
        f = c >> 5
        v = slab[pl.ds(c * LANES, LANES)]
        idxb[pl.ds(c * LANES, LANES)] = v.astype(jnp.int32) + f * VOCAB
        return carry

    lax.fori_loop(0, SP_CHUNKS, build, 0)

    pltpu.async_copy(table_hbm.at[idxb], vals, sem).wait()

    wvecs = [wv[pl.ds(d * LANES, LANES)] for d in range(DENSE_DIMS)]

    def accum(r, carry):
        o = r * LANES
        acc = vals[pl.ds(o, LANES)]
        for f in range(1, N_SPARSE):
            acc = acc + vals[pl.ds(f * RPW + o, LANES)]
        for d in range(DENSE_DIMS):
            acc = acc + slab[pl.ds((N_SPARSE + d) * RPW + o, LANES)] * wvecs[d]
        outv[pl.ds(o, LANES)] = acc
        return carry

    lax.fori_loop(0, CHUNKS, accum, 0)
    pltpu.sync_copy(outv, out_hbm.at[pl.ds(base, RPW)])


@functools.cache
def _build_sc_kernel():
    mesh = plsc.VectorSubcoreMesh(
        core_axis_name="c", subcore_axis_name="s",
        num_cores=NC, num_subcores=NSUB)
    return pl.kernel(
        _fieldwise_sc_body,
        out_type=jax.ShapeDtypeStruct((B,), jnp.float32),
        mesh=mesh,
        compiler_params=pltpu.CompilerParams(needs_layout_passes=False),
        scratch_types=[
            pltpu.VMEM((RPW * NCOL,), jnp.float32),
            pltpu.VMEM((N_SPARSE * RPW,), jnp.int32),
            pltpu.VMEM((N_SPARSE * RPW,), jnp.float32),
            pltpu.VMEM((RPW,), jnp.float32),
            pltpu.VMEM((DENSE_DIMS * LANES,), jnp.float32),
            pltpu.SemaphoreType.DMA,
        ],
    )


def kernel(raw_feat, sparse_tables, W_dense):
    raw_flat = raw_feat.T.reshape(B * NCOL)
    t2 = sparse_tables.reshape(N_SPARSE, VOCAB)
    t2 = lax.optimization_barrier(t2)
    table_flat = t2.reshape(N_SPARSE * VOCAB)
    w_rep = jnp.repeat(W_dense[0].astype(jnp.float32), LANES)
    return _build_sc_kernel()(raw_flat, table_flat, w_rep)


# speedup vs baseline: 2.6551x; 1.0585x over previous
import functools

import jax
import jax.numpy as jnp
from jax import lax
from jax.experimental import pallas as pl
from jax.experimental.pallas import tpu as pltpu
from jax.experimental.pallas import tpu_sc as plsc

B = 16384
N_SPARSE = 26
DENSE_DIMS = 13
NCOL = N_SPARSE + DENSE_DIMS
VOCAB = 100000
NC = 2
NSUB = 16
NW = NC * NSUB
RPW = B // NW
LANES = 16
CHUNKS = RPW // LANES
GROUPS = (0, 7, 14, 20, 26)
NG = len(GROUPS) - 1


def _fieldwise_sc_body(raw_hbm, table_hbm, w_hbm, out_hbm,
                       slab, idxb, vals, outv, wv, sem0, sem1):
    wid = lax.axis_index("s") * NC + lax.axis_index("c")
    base = wid * RPW
    sems = (sem0, sem1)
    descs = [
        pltpu.async_copy(raw_hbm.at[pl.ds(f * B + base, RPW)],
                         slab.at[pl.ds(f * RPW, RPW)], sem0)
        for f in range(NCOL)
    ]
    pltpu.sync_copy(w_hbm, wv)
    for d in descs:
        d.wait()

    def build(c, carry):
        f = c >> 5
        v = slab[pl.ds(c * LANES, LANES)]
        idxb[pl.ds(c * LANES, LANES)] = v.astype(jnp.int32) + f * VOCAB
        return carry

    def fire(g):
        lo, hi = GROUPS[g], GROUPS[g + 1]
        lax.fori_loop(lo * CHUNKS, hi * CHUNKS, build, 0)
        n = (hi - lo) * RPW
        return pltpu.async_copy(table_hbm.at[idxb.at[pl.ds(lo * RPW, n)]],
                                vals.at[pl.ds(lo * RPW, n)], sems[g % 2])

    gd = [None] * NG
    gd[0] = fire(0)
    gd[1] = fire(1)

    wvecs = [wv[pl.ds(d * LANES, LANES)] for d in range(DENSE_DIMS)]

    def dense(r, carry):
        o = r * LANES
        acc = slab[pl.ds(N_SPARSE * RPW + o, LANES)] * wvecs[0]
        for d in range(1, DENSE_DIMS):
            acc = acc + slab[pl.ds((N_SPARSE + d) * RPW + o, LANES)] * wvecs[d]
        outv[pl.ds(o, LANES)] = acc
        return carry

    lax.fori_loop(0, CHUNKS, dense, 0)

    def drain(g):
        gd[g].wait()
        lo, hi = GROUPS[g], GROUPS[g + 1]

        def acc_fn(r, carry):
            o = r * LANES
            acc = outv[pl.ds(o, LANES)]
            for f in range(lo, hi):
                acc = acc + vals[pl.ds(f * RPW + o, LANES)]
            outv[pl.ds(o, LANES)] = acc
            return carry

        lax.fori_loop(0, CHUNKS, acc_fn, 0)

    drain(0)
    gd[2] = fire(2)
    drain(1)
    gd[3] = fire(3)
    drain(2)
    drain(3)
    pltpu.sync_copy(outv, out_hbm.at[pl.ds(base, RPW)])


@functools.cache
def _build_sc_kernel():
    mesh = plsc.VectorSubcoreMesh(
        core_axis_name="c", subcore_axis_name="s",
        num_cores=NC, num_subcores=NSUB)
    return pl.kernel(
        _fieldwise_sc_body,
        out_type=jax.ShapeDtypeStruct((B,), jnp.float32),
        mesh=mesh,
        compiler_params=pltpu.CompilerParams(needs_layout_passes=False),
        scratch_types=[
            pltpu.VMEM((RPW * NCOL,), jnp.float32),
            pltpu.VMEM((N_SPARSE * RPW,), jnp.int32),
            pltpu.VMEM((N_SPARSE * RPW,), jnp.float32),
            pltpu.VMEM((RPW,), jnp.float32),
            pltpu.VMEM((DENSE_DIMS * LANES,), jnp.float32),
            pltpu.SemaphoreType.DMA,
            pltpu.SemaphoreType.DMA,
        ],
    )


def kernel(raw_feat, sparse_tables, W_dense):
    raw_flat = raw_feat.T.reshape(B * NCOL)
    t2 = sparse_tables.reshape(N_SPARSE, VOCAB)
    t2 = lax.optimization_barrier(t2)
    table_flat = t2.reshape(N_SPARSE * VOCAB)
    w_rep = jnp.repeat(W_dense[0].astype(jnp.float32), LANES)
    return _build_sc_kernel()(raw_flat, table_flat, w_rep)


# speedup vs baseline: 2.7794x; 1.0468x over previous
import functools

import jax
import jax.numpy as jnp
from jax import lax
from jax.experimental import pallas as pl
from jax.experimental.pallas import tpu as pltpu
from jax.experimental.pallas import tpu_sc as plsc

B = 16384
N_SPARSE = 26
DENSE_DIMS = 13
NCOL = N_SPARSE + DENSE_DIMS
VOCAB = 100000
NC = 2
NSUB = 16
NW = NC * NSUB
RPW = B // NW
LANES = 16
CHUNKS = RPW // LANES
GROUPS = (0, 7, 14, 20, 26)
NG = len(GROUPS) - 1


def _tc_prep_body(rawT_ref, w_ref, idx_ref, densep_ref):
    for f in range(N_SPARSE):
        idx_ref[pl.ds(f * B, B)] = rawT_ref[f, :].astype(jnp.int32) + f * VOCAB
    acc = rawT_ref[N_SPARSE, :] * w_ref[0, 0]
    for d in range(1, DENSE_DIMS):
        acc = acc + rawT_ref[N_SPARSE + d, :] * w_ref[0, d]
    densep_ref[...] = acc


@functools.cache
def _build_tc_prep():
    return pl.pallas_call(
        _tc_prep_body,
        out_shape=(jax.ShapeDtypeStruct((N_SPARSE * B,), jnp.int32),
                   jax.ShapeDtypeStruct((B,), jnp.float32)),
    )


def _sc_gather_body(idx_hbm, dp_hbm, table_hbm, out_hbm,
                    idxb, vals, outv, sem0, sem1):
    wid = lax.axis_index("s") * NC + lax.axis_index("c")
    base = wid * RPW
    sems = (sem0, sem1)
    idescs = [
        pltpu.async_copy(idx_hbm.at[pl.ds(f * B + base, RPW)],
                         idxb.at[pl.ds(f * RPW, RPW)], sem0)
        for f in range(N_SPARSE)
    ]
    dpd = pltpu.async_copy(dp_hbm.at[pl.ds(base, RPW)], outv, sem1)
    for d in idescs:
        d.wait()
    dpd.wait()

    def fire(g):
        lo, hi = GROUPS[g], GROUPS[g + 1]
        n = (hi - lo) * RPW
        return pltpu.async_copy(table_hbm.at[idxb.at[pl.ds(lo * RPW, n)]],
                                vals.at[pl.ds(lo * RPW, n)], sems[g % 2])

    gd = [None] * NG
    gd[0] = fire(0)
    gd[1] = fire(1)

    def drain(g):
        gd[g].wait()
        lo, hi = GROUPS[g], GROUPS[g + 1]

        def acc_fn(r, carry):
            o = r * LANES
            acc = outv[pl.ds(o, LANES)]
            for f in range(lo, hi):
                acc = acc + vals[pl.ds(f * RPW + o, LANES)]
            outv[pl.ds(o, LANES)] = acc
            return carry

        lax.fori_loop(0, CHUNKS, acc_fn, 0)

    drain(0)
    gd[2] = fire(2)
    drain(1)
    gd[3] = fire(3)
    drain(2)
    drain(3)
    pltpu.sync_copy(outv, out_hbm.at[pl.ds(base, RPW)])


@functools.cache
def _build_sc_gather():
    mesh = plsc.VectorSubcoreMesh(
        core_axis_name="c", subcore_axis_name="s",
        num_cores=NC, num_subcores=NSUB)
    return pl.kernel(
        _sc_gather_body,
        out_type=jax.ShapeDtypeStruct((B,), jnp.float32),
        mesh=mesh,
        compiler_params=pltpu.CompilerParams(needs_layout_passes=False),
        scratch_types=[
            pltpu.VMEM((N_SPARSE * RPW,), jnp.int32),
            pltpu.VMEM((N_SPARSE * RPW,), jnp.float32),
            pltpu.VMEM((RPW,), jnp.float32),
            pltpu.SemaphoreType.DMA,
            pltpu.SemaphoreType.DMA,
        ],
    )


def kernel(raw_feat, sparse_tables, W_dense):
    rawT = raw_feat.T
    idx_all, densep = _build_tc_prep()(rawT, W_dense)
    t2 = sparse_tables.reshape(N_SPARSE, VOCAB)
    t2 = lax.optimization_barrier(t2)
    table_flat = t2.reshape(N_SPARSE * VOCAB)
    return _build_sc_gather()(idx_all, densep, table_flat)
